# baseline (speedup 1.0000x reference)
import functools

import jax
import jax.numpy as jnp
from jax import lax
from jax.experimental import pallas as pl
from jax.experimental.pallas import tpu as pltpu
from jax.experimental.pallas import tpu_sc as plsc

B = 16384
D = 64
L = 16
NC = 2
NS = 16
NW = NC * NS
BW = B // NW
CH = 64
NCH = BW // CH


def _cf_body(uidx, iidx, u2, i2, out,
             uidx_v, iidx_v,
             ub0, ub1, ib0, ib1, out_v, sem):
    wid = lax.axis_index("s") * NC + lax.axis_index("c")
    base = wid * BW

    pltpu.sync_copy(uidx.at[pl.ds(base, BW)], uidx_v)
    pltpu.sync_copy(iidx.at[pl.ds(base, BW)], iidx_v)


    ubufs = (ub0, ub1)
    ibufs = (ib0, ib1)

    def fire(c):
        sl = pl.ds(c * CH, CH)
        pltpu.async_copy(u2.at[uidx_v.at[sl]], ubufs[c % 2], sem)
        pltpu.async_copy(i2.at[iidx_v.at[sl]], ibufs[c % 2], sem)

    def drain(c):
        sl = pl.ds(c * CH, CH)
        pltpu.make_async_copy(u2.at[uidx_v.at[sl]], ubufs[c % 2], sem).wait()
        pltpu.make_async_copy(i2.at[iidx_v.at[sl]], ibufs[c % 2], sem).wait()

    iota = lax.iota(jnp.int32, L)
    zero = jnp.zeros((L,), jnp.float32)

    def compute(c):
        ub = ubufs[c % 2]
        ib = ibufs[c % 2]

        def group(g, carry):
            b0 = c * CH + g * L
            res = zero
            for j in range(L):
                r = g * L + j
                acc = ub[r, pl.ds(0, L)] * ib[r, pl.ds(0, L)]
                for k in range(1, D // L):
                    acc = acc + (ub[r, pl.ds(k * L, L)] *
                                 ib[r, pl.ds(k * L, L)])
                res = jnp.where(iota == j, jnp.sum(acc), res)
            out_v[pl.ds(b0, L)] = res
            return carry

        lax.fori_loop(0, CH // L, group, 0)

    fire(0)
    for c in range(NCH):
        if c + 1 < NCH:
            fire(c + 1)
        drain(c)
        compute(c)

    pltpu.sync_copy(out_v, out.at[pl.ds(base, BW)])


_cf_kernel = functools.partial(
    pl.kernel,
    out_type=jax.ShapeDtypeStruct((B,), jnp.float32),
    mesh=plsc.VectorSubcoreMesh(core_axis_name="c", subcore_axis_name="s"),
    compiler_params=pltpu.CompilerParams(needs_layout_passes=False),
    scratch_types=[
        pltpu.VMEM((BW,), jnp.int32),
        pltpu.VMEM((BW,), jnp.int32),
        pltpu.VMEM((CH, 2 * D), jnp.float32),
        pltpu.VMEM((CH, 2 * D), jnp.float32),
        pltpu.VMEM((CH, 2 * D), jnp.float32),
        pltpu.VMEM((CH, 2 * D), jnp.float32),
        pltpu.VMEM((BW,), jnp.float32),
        pltpu.SemaphoreType.DMA,
    ],
)(_cf_body)


NROW = 1000000
TCB = 1024
TGRID = -(-NROW // TCB)


def _tr_body(i_ref, o_ref):
    o_ref[:, 0:D] = i_ref[...].T


_tr = pl.pallas_call(
    _tr_body,
    grid=(TGRID,),
    in_specs=[pl.BlockSpec((D, TCB), lambda g: (0, g))],
    out_specs=pl.BlockSpec((TCB, 2 * D), lambda g: (g, 0)),
    out_shape=jax.ShapeDtypeStruct((NROW, 2 * D), jnp.float32),
)


@jax.jit
def kernel(user_input, item_input, user_table, item_table):
    ui = user_input.astype(jnp.int32)
    ii = item_input.astype(jnp.int32)
    u2 = _tr(user_table.T)
    i2 = _tr(item_table.T)
    return _cf_kernel(ui, ii, u2, i2)


# speedup vs baseline: 2.7003x; 2.7003x over previous
import functools

import jax
import jax.numpy as jnp
from jax import lax
from jax.experimental import pallas as pl
from jax.experimental.pallas import tpu as pltpu
from jax.experimental.pallas import tpu_sc as plsc

B = 16384
D = 64
L = 16
NC = 2
NS = 16
NW = NC * NS
BW = B // NW
CH = 64
NCH = BW // CH


def _cf_body(uidx, iidx, u2, i2, out,
             uidx_v, iidx_v,
             ub0, ub1, ib0, ib1, out_v, sem):
    wid = lax.axis_index("s") * NC + lax.axis_index("c")
    base = wid * BW

    pltpu.sync_copy(uidx.at[pl.ds(base, BW)], uidx_v)
    pltpu.sync_copy(iidx.at[pl.ds(base, BW)], iidx_v)


    ubufs = (ub0, ub1)
    ibufs = (ib0, ib1)

    def fire(c):
        sl = pl.ds(c * CH, CH)
        pltpu.async_copy(u2.at[uidx_v.at[sl]], ubufs[c % 2], sem)
        pltpu.async_copy(i2.at[iidx_v.at[sl]], ibufs[c % 2], sem)

    def drain(c):
        sl = pl.ds(c * CH, CH)
        pltpu.make_async_copy(u2.at[uidx_v.at[sl]], ubufs[c % 2], sem).wait()
        pltpu.make_async_copy(i2.at[iidx_v.at[sl]], ibufs[c % 2], sem).wait()

    iota = lax.iota(jnp.int32, L)
    zero = jnp.zeros((L,), jnp.float32)

    def compute(c):
        ub = ubufs[c % 2]
        ib = ibufs[c % 2]

        def group(g, carry):
            b0 = c * CH + g * L
            res = zero
            for j in range(L):
                r = g * L + j
                acc = ub[r, pl.ds(0, L)] * ib[r, pl.ds(D, L)]
                for k in range(1, D // L):
                    acc = acc + (ub[r, pl.ds(k * L, L)] *
                                 ib[r, pl.ds(D + k * L, L)])
                res = jnp.where(iota == j, jnp.sum(acc), res)
            out_v[pl.ds(b0, L)] = res
            return carry

        lax.fori_loop(0, CH // L, group, 0)

    fire(0)
    for c in range(NCH):
        if c + 1 < NCH:
            fire(c + 1)
        drain(c)
        compute(c)

    pltpu.sync_copy(out_v, out.at[pl.ds(base, BW)])


_cf_kernel = functools.partial(
    pl.kernel,
    out_type=jax.ShapeDtypeStruct((B,), jnp.float32),
    mesh=plsc.VectorSubcoreMesh(core_axis_name="c", subcore_axis_name="s"),
    compiler_params=pltpu.CompilerParams(needs_layout_passes=False),
    scratch_types=[
        pltpu.VMEM((BW,), jnp.int32),
        pltpu.VMEM((BW,), jnp.int32),
        pltpu.VMEM((CH, 2 * D), jnp.float32),
        pltpu.VMEM((CH, 2 * D), jnp.float32),
        pltpu.VMEM((CH, 2 * D), jnp.float32),
        pltpu.VMEM((CH, 2 * D), jnp.float32),
        pltpu.VMEM((BW,), jnp.float32),
        pltpu.SemaphoreType.DMA,
    ],
)(_cf_body)


NROW = 1000000
TCB = 4096
TGRID = -(-NROW // TCB)


def _tr_body(u_ref, i_ref, o_ref):
    o_ref[:, 0:D] = u_ref[...].T
    o_ref[:, D:2 * D] = i_ref[...].T


_tr = pl.pallas_call(
    _tr_body,
    grid=(TGRID,),
    in_specs=[pl.BlockSpec((D, TCB), lambda g: (0, g)),
              pl.BlockSpec((D, TCB), lambda g: (0, g))],
    out_specs=pl.BlockSpec((TCB, 2 * D), lambda g: (g, 0)),
    out_shape=jax.ShapeDtypeStruct((NROW, 2 * D), jnp.float32),
)


@jax.jit
def kernel(user_input, item_input, user_table, item_table):
    ui = user_input.astype(jnp.int32)
    ii = item_input.astype(jnp.int32)
    c2 = _tr(user_table.T, item_table.T)
    return _cf_kernel(ui, ii, c2, c2)


# speedup vs baseline: 3.0992x; 1.1477x over previous
import functools

import jax
import jax.numpy as jnp
from jax import lax
from jax.experimental import pallas as pl
from jax.experimental.pallas import tpu as pltpu
from jax.experimental.pallas import tpu_sc as plsc

B = 16384
D = 64
L = 16
NC = 2
NS = 16
NW = NC * NS
BW = B // NW
CH = 64
NCH = BW // CH


def _cf_body(uidx, iidx, u2, i2, out,
             uidx_v, iidx_v,
             ub0, ub1, ib0, ib1, out_v, sem):
    wid = lax.axis_index("s") * NC + lax.axis_index("c")
    base = wid * BW

    pltpu.sync_copy(uidx.at[pl.ds(base, BW)], uidx_v)
    pltpu.sync_copy(iidx.at[pl.ds(base, BW)], iidx_v)


    ubufs = (ub0, ub1)
    ibufs = (ib0, ib1)

    def fire(c):
        sl = pl.ds(c * CH, CH)
        pltpu.async_copy(u2.at[uidx_v.at[sl]], ubufs[c % 2], sem)
        pltpu.async_copy(i2.at[iidx_v.at[sl]], ibufs[c % 2], sem)

    def drain(c):
        sl = pl.ds(c * CH, CH)
        pltpu.make_async_copy(u2.at[uidx_v.at[sl]], ubufs[c % 2], sem).wait()
        pltpu.make_async_copy(i2.at[iidx_v.at[sl]], ibufs[c % 2], sem).wait()

    iota = lax.iota(jnp.int32, L)
    zero = jnp.zeros((L,), jnp.float32)

    def compute(c):
        ub = ubufs[c % 2]
        ib = ibufs[c % 2]

        def group(g, carry):
            b0 = c * CH + g * L
            res = zero
            for j in range(L):
                r = g * L + j
                acc = ub[r, pl.ds(0, L)] * ib[r, pl.ds(D, L)]
                for k in range(1, D // L):
                    acc = acc + (ub[r, pl.ds(k * L, L)] *
                                 ib[r, pl.ds(D + k * L, L)])
                res = jnp.where(iota == j, jnp.sum(acc), res)
            out_v[pl.ds(b0, L)] = res
            return carry

        lax.fori_loop(0, CH // L, group, 0)

    fire(0)
    for c in range(NCH):
        if c + 1 < NCH:
            fire(c + 1)
        drain(c)
        compute(c)

    pltpu.sync_copy(out_v, out.at[pl.ds(base, BW)])


_cf_kernel = functools.partial(
    pl.kernel,
    out_type=jax.ShapeDtypeStruct((B,), jnp.float32),
    mesh=plsc.VectorSubcoreMesh(core_axis_name="c", subcore_axis_name="s"),
    compiler_params=pltpu.CompilerParams(needs_layout_passes=False),
    scratch_types=[
        pltpu.VMEM((BW,), jnp.int32),
        pltpu.VMEM((BW,), jnp.int32),
        pltpu.VMEM((CH, 2 * D), jnp.float32),
        pltpu.VMEM((CH, 2 * D), jnp.float32),
        pltpu.VMEM((CH, 2 * D), jnp.float32),
        pltpu.VMEM((CH, 2 * D), jnp.float32),
        pltpu.VMEM((BW,), jnp.float32),
        pltpu.SemaphoreType.DMA,
    ],
)(_cf_body)


NROW = 1000000
TCB = 8192
TGRID = -(-NROW // TCB)


def _tr_body(u_ref, i_ref, o_ref):
    o_ref[:, 0:D] = u_ref[...].T
    o_ref[:, D:2 * D] = i_ref[...].T


_tr = pl.pallas_call(
    _tr_body,
    grid=(TGRID,),
    in_specs=[pl.BlockSpec((D, TCB), lambda g: (0, g)),
              pl.BlockSpec((D, TCB), lambda g: (0, g))],
    out_specs=pl.BlockSpec((TCB, 2 * D), lambda g: (g, 0)),
    out_shape=jax.ShapeDtypeStruct((NROW, 2 * D), jnp.float32),
)


@jax.jit
def kernel(user_input, item_input, user_table, item_table):
    ui = user_input.astype(jnp.int32)
    ii = item_input.astype(jnp.int32)
    c2 = _tr(user_table.T, item_table.T)
    return _cf_kernel(ui, ii, c2, c2)


# speedup vs baseline: 3.3063x; 1.0669x over previous
import functools

import jax
import jax.numpy as jnp
from jax import lax
from jax.experimental import pallas as pl
from jax.experimental.pallas import tpu as pltpu
from jax.experimental.pallas import tpu_sc as plsc

B = 16384
D = 64
L = 16
NC = 2
NS = 16
NW = NC * NS
BW = B // NW
CH = 64
NCH = BW // CH


def _cf_body(uidx, iidx, u2, i2, out,
             uidx_v, iidx_v,
             ub0, ub1, ib0, ib1, out_v, sem):
    wid = lax.axis_index("s") * NC + lax.axis_index("c")
    base = wid * BW

    pltpu.sync_copy(uidx.at[pl.ds(base, BW)], uidx_v)
    pltpu.sync_copy(iidx.at[pl.ds(base, BW)], iidx_v)


    ubufs = (ub0, ub1)
    ibufs = (ib0, ib1)

    def fire(c):
        sl = pl.ds(c * CH, CH)
        pltpu.async_copy(u2.at[uidx_v.at[sl]], ubufs[c % 2], sem)
        pltpu.async_copy(i2.at[iidx_v.at[sl]], ibufs[c % 2], sem)

    def drain(c):
        sl = pl.ds(c * CH, CH)
        pltpu.make_async_copy(u2.at[uidx_v.at[sl]], ubufs[c % 2], sem).wait()
        pltpu.make_async_copy(i2.at[iidx_v.at[sl]], ibufs[c % 2], sem).wait()

    iota = lax.iota(jnp.int32, L)
    zero = jnp.zeros((L,), jnp.float32)

    def compute(c):
        ub = ubufs[c % 2]
        ib = ibufs[c % 2]

        def group(g, carry):
            b0 = c * CH + g * L
            res = zero
            for j in range(L):
                r = g * L + j
                acc = ub[r, pl.ds(0, L)] * ib[r, pl.ds(D, L)]
                for k in range(1, D // L):
                    acc = acc + (ub[r, pl.ds(k * L, L)] *
                                 ib[r, pl.ds(D + k * L, L)])
                res = jnp.where(iota == j, jnp.sum(acc), res)
            out_v[pl.ds(b0, L)] = res
            return carry

        lax.fori_loop(0, CH // L, group, 0)

    fire(0)
    for c in range(NCH):
        if c + 1 < NCH:
            fire(c + 1)
        drain(c)
        compute(c)

    pltpu.sync_copy(out_v, out.at[pl.ds(base, BW)])


_cf_kernel = functools.partial(
    pl.kernel,
    out_type=jax.ShapeDtypeStruct((B,), jnp.float32),
    mesh=plsc.VectorSubcoreMesh(core_axis_name="c", subcore_axis_name="s"),
    compiler_params=pltpu.CompilerParams(needs_layout_passes=False),
    scratch_types=[
        pltpu.VMEM((BW,), jnp.int32),
        pltpu.VMEM((BW,), jnp.int32),
        pltpu.VMEM((CH, 2 * D), jnp.float32),
        pltpu.VMEM((CH, 2 * D), jnp.float32),
        pltpu.VMEM((CH, 2 * D), jnp.float32),
        pltpu.VMEM((CH, 2 * D), jnp.float32),
        pltpu.VMEM((BW,), jnp.float32),
        pltpu.SemaphoreType.DMA,
    ],
)(_cf_body)


NROW = 1000000
TCB = 16384
TGRID = -(-NROW // TCB)


def _tr_body(u_ref, i_ref, o_ref):
    o_ref[:, 0:D] = u_ref[...].T
    o_ref[:, D:2 * D] = i_ref[...].T


_tr = pl.pallas_call(
    _tr_body,
    grid=(TGRID,),
    in_specs=[pl.BlockSpec((D, TCB), lambda g: (0, g)),
              pl.BlockSpec((D, TCB), lambda g: (0, g))],
    out_specs=pl.BlockSpec((TCB, 2 * D), lambda g: (g, 0)),
    out_shape=jax.ShapeDtypeStruct((NROW, 2 * D), jnp.float32),
)


@jax.jit
def kernel(user_input, item_input, user_table, item_table):
    ui = user_input.astype(jnp.int32)
    ii = item_input.astype(jnp.int32)
    c2 = _tr(user_table.T, item_table.T)
    return _cf_kernel(ui, ii, c2, c2)


# speedup vs baseline: 3.3352x; 1.0087x over previous
import functools

import jax
import jax.numpy as jnp
from jax import lax
from jax.experimental import pallas as pl
from jax.experimental.pallas import tpu as pltpu
from jax.experimental.pallas import tpu_sc as plsc

B = 16384
D = 64
L = 16
NC = 2
NS = 16
NW = NC * NS
BW = B // NW
CH = 64
NCH = BW // CH


def _cf_body(uidx, iidx, u2, i2, out,
             uidx_v, iidx_v,
             ub0, ub1, ib0, ib1, out_v, sem):
    wid = lax.axis_index("s") * NC + lax.axis_index("c")
    base = wid * BW

    pltpu.sync_copy(uidx.at[pl.ds(base, BW)], uidx_v)
    pltpu.sync_copy(iidx.at[pl.ds(base, BW)], iidx_v)


    ubufs = (ub0, ub1)
    ibufs = (ib0, ib1)

    def fire(c):
        sl = pl.ds(c * CH, CH)
        pltpu.async_copy(u2.at[uidx_v.at[sl]], ubufs[c % 2], sem)
        pltpu.async_copy(i2.at[iidx_v.at[sl]], ibufs[c % 2], sem)

    def drain(c):
        sl = pl.ds(c * CH, CH)
        pltpu.make_async_copy(u2.at[uidx_v.at[sl]], ubufs[c % 2], sem).wait()
        pltpu.make_async_copy(i2.at[iidx_v.at[sl]], ibufs[c % 2], sem).wait()

    iota = lax.iota(jnp.int32, L)
    zero = jnp.zeros((L,), jnp.float32)

    def compute(c):
        ub = ubufs[c % 2]
        ib = ibufs[c % 2]

        def group(g, carry):
            b0 = c * CH + g * L
            res = zero
            for j in range(L):
                r = g * L + j
                acc = ub[r, pl.ds(0, L)] * ib[r, pl.ds(D, L)]
                for k in range(1, D // L):
                    acc = acc + (ub[r, pl.ds(k * L, L)] *
                                 ib[r, pl.ds(D + k * L, L)])
                res = jnp.where(iota == j, jnp.sum(acc), res)
            out_v[pl.ds(b0, L)] = res
            return carry

        lax.fori_loop(0, CH // L, group, 0)

    fire(0)
    for c in range(NCH):
        if c + 1 < NCH:
            fire(c + 1)
        drain(c)
        compute(c)

    pltpu.sync_copy(out_v, out.at[pl.ds(base, BW)])


_cf_kernel = functools.partial(
    pl.kernel,
    out_type=jax.ShapeDtypeStruct((B,), jnp.float32),
    mesh=plsc.VectorSubcoreMesh(core_axis_name="c", subcore_axis_name="s"),
    compiler_params=pltpu.CompilerParams(needs_layout_passes=False),
    scratch_types=[
        pltpu.VMEM((BW,), jnp.int32),
        pltpu.VMEM((BW,), jnp.int32),
        pltpu.VMEM((CH, 2 * D), jnp.float32),
        pltpu.VMEM((CH, 2 * D), jnp.float32),
        pltpu.VMEM((CH, 2 * D), jnp.float32),
        pltpu.VMEM((CH, 2 * D), jnp.float32),
        pltpu.VMEM((BW,), jnp.float32),
        pltpu.SemaphoreType.DMA,
    ],
)(_cf_body)


NROW = 1000000
TCB = 20480
TGRID = -(-NROW // TCB)


def _tr_body(u_ref, i_ref, o_ref):
    o_ref[:, 0:D] = u_ref[...].T
    o_ref[:, D:2 * D] = i_ref[...].T


_tr = pl.pallas_call(
    _tr_body,
    grid=(TGRID,),
    in_specs=[pl.BlockSpec((D, TCB), lambda g: (0, g)),
              pl.BlockSpec((D, TCB), lambda g: (0, g))],
    out_specs=pl.BlockSpec((TCB, 2 * D), lambda g: (g, 0)),
    out_shape=jax.ShapeDtypeStruct((NROW, 2 * D), jnp.float32),
)


@jax.jit
def kernel(user_input, item_input, user_table, item_table):
    ui = user_input.astype(jnp.int32)
    ii = item_input.astype(jnp.int32)
    c2 = _tr(user_table.T, item_table.T)
    return _cf_kernel(ui, ii, c2, c2)
